# baseline (speedup 1.0000x reference)
import functools

import jax
import jax.numpy as jnp
from jax.experimental import pallas as pl
from jax.experimental.pallas import tpu as pltpu

_POS_T = 0.7
_NEG_T = 0.3
_KPOS = 128
_KNEG = 256
_R = 160
_C = 128
_NPAD = _R * _C
_CHUNK = 32


def _body(a_ref, gt_ref, lab_ref, labout_ref, bboxout_ref,
          miou_s, cls_s, enc_s, keys_s, *, n_valid, n_gt):
    b = pl.program_id(0)

    def chunk(c, _):
        rows = pl.ds(c * _CHUNK, _CHUNK)
        ax1 = a_ref[0, rows, :]
        ay1 = a_ref[1, rows, :]
        ax2 = a_ref[2, rows, :]
        ay2 = a_ref[3, rows, :]
        area_a = jnp.maximum(ax2 - ax1, 0.0) * jnp.maximum(ay2 - ay1, 0.0)

        def gstep(g, carry):
            best, bm1, bm2, bm3, bm4, blab = carry
            gx1 = gt_ref[b, g, 0]
            gy1 = gt_ref[b, g, 1]
            gx2 = gt_ref[b, g, 2]
            gy2 = gt_ref[b, g, 3]
            area_g = jnp.maximum(gx2 - gx1, 0.0) * jnp.maximum(gy2 - gy1, 0.0)
            ltx = jnp.maximum(ax1, gx1)
            lty = jnp.maximum(ay1, gy1)
            rbx = jnp.minimum(ax2, gx2)
            rby = jnp.minimum(ay2, gy2)
            w = jnp.maximum(rbx - ltx, 0.0)
            h = jnp.maximum(rby - lty, 0.0)
            inter = w * h
            iou = inter / (area_a + area_g - inter + 1e-9)
            lb = lab_ref[b, g]
            better = iou > best
            best = jnp.where(better, iou, best)
            bm1 = jnp.where(better, gx1, bm1)
            bm2 = jnp.where(better, gy1, bm2)
            bm3 = jnp.where(better, gx2, bm3)
            bm4 = jnp.where(better, gy2, bm4)
            blab = jnp.where(better, lb, blab)
            return best, bm1, bm2, bm3, bm4, blab

        zf = jnp.zeros((_CHUNK, _C), jnp.float32)
        init = (jnp.full((_CHUNK, _C), -1.0, jnp.float32), zf, zf, zf, zf,
                jnp.zeros((_CHUNK, _C), jnp.int32))
        best, m1, m2, m3, m4, mlab = jax.lax.fori_loop(0, n_gt, gstep, init)

        miou_s[rows, :] = best
        cls_s[rows, :] = mlab

        aw = jnp.maximum(ax2 - ax1, 1e-3)
        ah = jnp.maximum(ay2 - ay1, 1e-3)
        axc = (ax1 + ax2) * 0.5
        ayc = (ay1 + ay2) * 0.5
        gw = jnp.maximum(m3 - m1, 1e-3)
        gh = jnp.maximum(m4 - m2, 1e-3)
        gxc = (m1 + m3) * 0.5
        gyc = (m2 + m4) * 0.5
        enc_s[0, rows, :] = (gxc - axc) / aw
        enc_s[1, rows, :] = (gyc - ayc) / ah
        enc_s[2, rows, :] = jnp.log(gw / aw)
        enc_s[3, rows, :] = jnp.log(gh / ah)

        ridx = jax.lax.broadcasted_iota(jnp.int32, (_CHUNK, _C), 0) + c * _CHUNK
        aidx = ridx * _C + jax.lax.broadcasted_iota(jnp.int32, (_CHUNK, _C), 1)
        valid = aidx < n_valid
        bits = jax.lax.bitcast_convert_type(best, jnp.int32)
        posk = jnp.where(valid & (best >= _POS_T), bits, 0)
        negk = jnp.where(valid & (best <= _NEG_T), jnp.int32(0x7FFFFFFF) - bits, 0)
        keys_s[0, rows, :] = posk
        keys_s[1, rows, :] = negk
        return 0

    jax.lax.fori_loop(0, _R // _CHUNK, chunk, 0)

    def bit_step(i, ts):
        tp, tn = ts
        m = jnp.left_shift(jnp.int32(1), 30 - i)
        tp2 = jnp.bitwise_or(tp, m)
        tn2 = jnp.bitwise_or(tn, m)
        cp = jnp.sum((keys_s[0] >= tp2).astype(jnp.int32))
        cn = jnp.sum((keys_s[1] >= tn2).astype(jnp.int32))
        tp = jnp.where(cp >= _KPOS, tp2, tp)
        tn = jnp.where(cn >= _KNEG, tn2, tn)
        return tp, tn

    tpos, tneg = jax.lax.fori_loop(
        0, 31, bit_step, (jnp.int32(0), jnp.int32(0)))

    upper = (jax.lax.broadcasted_iota(jnp.int32, (_C, _C), 0)
             <= jax.lax.broadcasted_iota(jnp.int32, (_C, _C), 1)
             ).astype(jnp.float32)
    lstrict = (jax.lax.broadcasted_iota(jnp.int32, (_R, _R), 1)
               < jax.lax.broadcasted_iota(jnp.int32, (_R, _R), 0)
               ).astype(jnp.float32)

    def select(key, thresh, k):
        gtm = key > thresh
        eqm = key == thresh
        ngt = jnp.sum(gtm.astype(jnp.int32))
        r = (k - ngt).astype(jnp.float32)
        eqf = eqm.astype(jnp.float32)
        m1 = jnp.dot(eqf, upper, preferred_element_type=jnp.float32)
        t2 = jnp.dot(lstrict, m1, preferred_element_type=jnp.float32)
        cum = m1 + t2[:, _C - 1:_C]
        return gtm | (eqm & (key > 0) & (cum <= r))

    kpos = select(keys_s[0], tpos, _KPOS)
    kneg = select(keys_s[1], tneg, _KNEG)

    lab = jnp.where(kpos, cls_s[:, :],
                    jnp.where(kneg, jnp.int32(0), jnp.int32(-1)))
    labout_ref[0] = lab
    pm = kpos.astype(jnp.float32)
    bboxout_ref[0, 0] = enc_s[0] * pm
    bboxout_ref[0, 1] = enc_s[1] * pm
    bboxout_ref[0, 2] = enc_s[2] * pm
    bboxout_ref[0, 3] = enc_s[3] * pm


def kernel(anchors, batch_gt_boxes, batch_labels):
    n = anchors.shape[0]
    bsz, n_gt = batch_labels.shape
    a_pad = jnp.pad(anchors, ((0, _NPAD - n), (0, 0)))
    a_t = a_pad.T.reshape(4, _R, _C)
    labels_p, bbox_p = pl.pallas_call(
        functools.partial(_body, n_valid=n, n_gt=n_gt),
        grid=(bsz,),
        in_specs=[
            pl.BlockSpec((4, _R, _C), lambda b: (0, 0, 0)),
            pl.BlockSpec(memory_space=pltpu.SMEM),
            pl.BlockSpec(memory_space=pltpu.SMEM),
        ],
        out_specs=[
            pl.BlockSpec((1, _R, _C), lambda b: (b, 0, 0)),
            pl.BlockSpec((1, 4, _R, _C), lambda b: (b, 0, 0, 0)),
        ],
        out_shape=[
            jax.ShapeDtypeStruct((bsz, _R, _C), jnp.int32),
            jax.ShapeDtypeStruct((bsz, 4, _R, _C), jnp.float32),
        ],
        scratch_shapes=[
            pltpu.VMEM((_R, _C), jnp.float32),
            pltpu.VMEM((_R, _C), jnp.int32),
            pltpu.VMEM((4, _R, _C), jnp.float32),
            pltpu.VMEM((2, _R, _C), jnp.int32),
        ],
        compiler_params=pltpu.CompilerParams(
            dimension_semantics=("parallel",)),
    )(a_t, batch_gt_boxes, batch_labels)
    labels = labels_p.reshape(bsz, _NPAD)[:, :n]
    bbox = bbox_p.reshape(bsz, 4, _NPAD)[:, :, :n].transpose(0, 2, 1)
    return labels, bbox


# speedup vs baseline: 1.0346x; 1.0346x over previous
import functools

import jax
import jax.numpy as jnp
from jax.experimental import pallas as pl
from jax.experimental.pallas import tpu as pltpu

_POS_T = 0.7
_NEG_T = 0.3
_KPOS = 128
_KNEG = 256
_R = 160
_C = 128
_NPAD = _R * _C
_CHUNK = 40


def _body(a_ref, gt_ref, lab_ref, labout_ref, bboxout_ref,
          miou_s, cls_s, enc_s, keys_s, *, n_valid, n_gt):
    b = pl.program_id(0)

    def chunk(c, _):
        rows = pl.ds(c * _CHUNK, _CHUNK)
        ax1 = a_ref[0, rows, :]
        ay1 = a_ref[1, rows, :]
        ax2 = a_ref[2, rows, :]
        ay2 = a_ref[3, rows, :]
        area_a = jnp.maximum(ax2 - ax1, 0.0) * jnp.maximum(ay2 - ay1, 0.0)

        def gstep(g, carry):
            best, bm1, bm2, bm3, bm4, blab = carry
            gx1 = gt_ref[b, g, 0]
            gy1 = gt_ref[b, g, 1]
            gx2 = gt_ref[b, g, 2]
            gy2 = gt_ref[b, g, 3]
            area_g = jnp.maximum(gx2 - gx1, 0.0) * jnp.maximum(gy2 - gy1, 0.0)
            ltx = jnp.maximum(ax1, gx1)
            lty = jnp.maximum(ay1, gy1)
            rbx = jnp.minimum(ax2, gx2)
            rby = jnp.minimum(ay2, gy2)
            w = jnp.maximum(rbx - ltx, 0.0)
            h = jnp.maximum(rby - lty, 0.0)
            inter = w * h
            iou = inter / (area_a + area_g - inter + 1e-9)
            lb = lab_ref[b, g]
            better = iou > best
            best = jnp.where(better, iou, best)
            bm1 = jnp.where(better, gx1, bm1)
            bm2 = jnp.where(better, gy1, bm2)
            bm3 = jnp.where(better, gx2, bm3)
            bm4 = jnp.where(better, gy2, bm4)
            blab = jnp.where(better, lb, blab)
            return best, bm1, bm2, bm3, bm4, blab

        zf = jnp.zeros((_CHUNK, _C), jnp.float32)
        init = (jnp.full((_CHUNK, _C), -1.0, jnp.float32), zf, zf, zf, zf,
                jnp.zeros((_CHUNK, _C), jnp.int32))
        best, m1, m2, m3, m4, mlab = jax.lax.fori_loop(0, n_gt, gstep, init)

        miou_s[rows, :] = best
        cls_s[rows, :] = mlab

        aw = jnp.maximum(ax2 - ax1, 1e-3)
        ah = jnp.maximum(ay2 - ay1, 1e-3)
        axc = (ax1 + ax2) * 0.5
        ayc = (ay1 + ay2) * 0.5
        gw = jnp.maximum(m3 - m1, 1e-3)
        gh = jnp.maximum(m4 - m2, 1e-3)
        gxc = (m1 + m3) * 0.5
        gyc = (m2 + m4) * 0.5
        enc_s[0, rows, :] = (gxc - axc) / aw
        enc_s[1, rows, :] = (gyc - ayc) / ah
        enc_s[2, rows, :] = jnp.log(gw / aw)
        enc_s[3, rows, :] = jnp.log(gh / ah)

        ridx = jax.lax.broadcasted_iota(jnp.int32, (_CHUNK, _C), 0) + c * _CHUNK
        aidx = ridx * _C + jax.lax.broadcasted_iota(jnp.int32, (_CHUNK, _C), 1)
        valid = aidx < n_valid
        bits = jax.lax.bitcast_convert_type(best, jnp.int32)
        posk = jnp.where(valid & (best >= _POS_T), bits, 0)
        negk = jnp.where(valid & (best <= _NEG_T), jnp.int32(0x7FFFFFFF) - bits, 0)
        keys_s[0, rows, :] = posk
        keys_s[1, rows, :] = negk
        return 0

    jax.lax.fori_loop(0, _R // _CHUNK, chunk, 0)

    def bit_step(i, ts):
        tp, tn = ts
        m = jnp.left_shift(jnp.int32(1), 30 - i)
        tp2 = jnp.bitwise_or(tp, m)
        tn2 = jnp.bitwise_or(tn, m)
        cp = jnp.sum((keys_s[0] >= tp2).astype(jnp.int32))
        cn = jnp.sum((keys_s[1] >= tn2).astype(jnp.int32))
        tp = jnp.where(cp >= _KPOS, tp2, tp)
        tn = jnp.where(cn >= _KNEG, tn2, tn)
        return tp, tn

    tpos, tneg = jax.lax.fori_loop(
        0, 31, bit_step, (jnp.int32(0), jnp.int32(0)))

    upper = (jax.lax.broadcasted_iota(jnp.int32, (_C, _C), 0)
             <= jax.lax.broadcasted_iota(jnp.int32, (_C, _C), 1)
             ).astype(jnp.float32)
    lstrict = (jax.lax.broadcasted_iota(jnp.int32, (_R, _R), 1)
               < jax.lax.broadcasted_iota(jnp.int32, (_R, _R), 0)
               ).astype(jnp.float32)

    def select(key, thresh, k):
        gtm = key > thresh
        eqm = key == thresh
        ngt = jnp.sum(gtm.astype(jnp.int32))
        r = (k - ngt).astype(jnp.float32)
        eqf = eqm.astype(jnp.float32)
        m1 = jnp.dot(eqf, upper, preferred_element_type=jnp.float32)
        t2 = jnp.dot(lstrict, m1, preferred_element_type=jnp.float32)
        cum = m1 + t2[:, _C - 1:_C]
        return gtm | (eqm & (key > 0) & (cum <= r))

    kpos = select(keys_s[0], tpos, _KPOS)
    kneg = select(keys_s[1], tneg, _KNEG)

    lab = jnp.where(kpos, cls_s[:, :],
                    jnp.where(kneg, jnp.int32(0), jnp.int32(-1)))
    labout_ref[0] = lab
    pm = kpos.astype(jnp.float32)
    bboxout_ref[0, 0] = enc_s[0] * pm
    bboxout_ref[0, 1] = enc_s[1] * pm
    bboxout_ref[0, 2] = enc_s[2] * pm
    bboxout_ref[0, 3] = enc_s[3] * pm


def kernel(anchors, batch_gt_boxes, batch_labels):
    n = anchors.shape[0]
    bsz, n_gt = batch_labels.shape
    a_pad = jnp.pad(anchors, ((0, _NPAD - n), (0, 0)))
    a_t = a_pad.T.reshape(4, _R, _C)
    labels_p, bbox_p = pl.pallas_call(
        functools.partial(_body, n_valid=n, n_gt=n_gt),
        grid=(bsz,),
        in_specs=[
            pl.BlockSpec((4, _R, _C), lambda b: (0, 0, 0)),
            pl.BlockSpec(memory_space=pltpu.SMEM),
            pl.BlockSpec(memory_space=pltpu.SMEM),
        ],
        out_specs=[
            pl.BlockSpec((1, _R, _C), lambda b: (b, 0, 0)),
            pl.BlockSpec((1, 4, _R, _C), lambda b: (b, 0, 0, 0)),
        ],
        out_shape=[
            jax.ShapeDtypeStruct((bsz, _R, _C), jnp.int32),
            jax.ShapeDtypeStruct((bsz, 4, _R, _C), jnp.float32),
        ],
        scratch_shapes=[
            pltpu.VMEM((_R, _C), jnp.float32),
            pltpu.VMEM((_R, _C), jnp.int32),
            pltpu.VMEM((4, _R, _C), jnp.float32),
            pltpu.VMEM((2, _R, _C), jnp.int32),
        ],
        compiler_params=pltpu.CompilerParams(
            dimension_semantics=("parallel",)),
    )(a_t, batch_gt_boxes, batch_labels)
    labels = labels_p.reshape(bsz, _NPAD)[:, :n]
    bbox = bbox_p.reshape(bsz, 4, _NPAD)[:, :, :n].transpose(0, 2, 1)
    return labels, bbox
